# Initial kernel scaffold; baseline (speedup 1.0000x reference)
import functools

import jax
import jax.numpy as jnp
from jax import lax
from jax.experimental import pallas as pl
from jax.experimental.pallas import tpu as pltpu
from jax.experimental.pallas import tpu_sc as plsc

N_NODES = 10000
N_EDGES = 320000
D = 128

_info = plsc.get_sparse_core_info()
NC = _info.num_cores
NS = _info.num_subcores
L = _info.num_lanes
NW = NC * NS

CHUNK = 128
EPW = -(-N_EDGES // (NW * CHUNK)) * CHUNK
EPAD = EPW * NW
NCHUNK = EPW // CHUNK

ACC_ROWS = 10240
ZROWS = ACC_ROWS // NS
WB = N_NODES // NS


def _elu_body(x_ref, w_ref, o_ref):
    j = pl.program_id(1)
    y = x_ref[...] * w_ref[...]
    y = jnp.where(y > 0, y, jnp.expm1(y))
    o_ref[...] = y * (1.0 + j.astype(jnp.float32))


def _build_table(graph_embedding, W):
    blk = 1000
    grid = (N_NODES // blk, 2)
    return pl.pallas_call(
        _elu_body,
        grid=grid,
        in_specs=[
            pl.BlockSpec((blk, D), lambda i, j: (i, 0)),
            pl.BlockSpec((1, D), lambda i, j: (0, 0)),
        ],
        out_specs=pl.BlockSpec((blk, D), lambda i, j: (j * (N_NODES // blk) + i, 0)),
        out_shape=jax.ShapeDtypeStruct((2 * N_NODES, D), jnp.float32),
    )(graph_embedding, W)


_mesh = plsc.VectorSubcoreMesh(core_axis_name="c", subcore_axis_name="s")


@functools.partial(
    pl.kernel,
    mesh=_mesh,
    out_type=jax.ShapeDtypeStruct((NC, N_NODES, D), jnp.float32),
    scratch_types=[
        pltpu.VMEM((3, CHUNK), jnp.int32),
        pltpu.VMEM((CHUNK,), jnp.int32),
        pltpu.VMEM((CHUNK, D), jnp.float32),
        pltpu.VMEM((16, D), jnp.float32),
        pltpu.VMEM_SHARED((ACC_ROWS, D), jnp.float32),
        pltpu.SemaphoreType.DMA,
    ],
)
def _sc_gather_scatter(tbl_hbm, stk_hbm, out_hbm,
                       stk_v, gidx_v, rows_v, zb_v, acc_sh, sem):
    c = lax.axis_index("c")
    s = lax.axis_index("s")

    zero16 = jnp.zeros((L,), jnp.float32)
    for r in range(16):
        for k in range(D // L):
            zb_v[r, pl.ds(k * L, L)] = zero16
    base_row = s * ZROWS
    for t in range(ZROWS // 16):
        pltpu.sync_copy(zb_v, acc_sh.at[pl.ds(base_row + t * 16, 16)])
    plsc.subcore_barrier()

    w = c * NS + s
    ebase = w * EPW

    def body(j, carry):
        pltpu.sync_copy(stk_hbm.at[:, pl.ds(ebase + j * CHUNK, CHUNK)], stk_v)
        for k in range(CHUNK // L):
            sl = pl.ds(k * L, L)
            s16 = stk_v[0, sl]
            e16 = stk_v[2, sl]
            gidx_v[sl] = s16 + (1 - (e16 & 1)) * N_NODES
        pltpu.async_copy(tbl_hbm.at[gidx_v], rows_v, sem).wait()
        pltpu.sync_copy(rows_v, acc_sh.at[stk_v.at[1]], add=True)
        return carry

    lax.fori_loop(0, NCHUNK, body, 0)
    plsc.subcore_barrier()
    pltpu.sync_copy(acc_sh.at[pl.ds(s * WB, WB)],
                    out_hbm.at[c, pl.ds(s * WB, WB)])


def _add_body(p_ref, o_ref):
    o_ref[...] = p_ref[0] + p_ref[1]


def _sum_partials(partials):
    blk = 1000
    return pl.pallas_call(
        _add_body,
        grid=(N_NODES // blk,),
        in_specs=[pl.BlockSpec((2, blk, D), lambda i: (0, i, 0))],
        out_specs=pl.BlockSpec((blk, D), lambda i: (i, 0)),
        out_shape=jax.ShapeDtypeStruct((N_NODES, D), jnp.float32),
    )(partials)


def kernel(graph_embedding, edge_index, e_feat, W):
    tbl = _build_table(graph_embedding, W)

    src = edge_index[0].astype(jnp.int32)
    dst = edge_index[1].astype(jnp.int32)
    e = e_feat.astype(jnp.int32)
    pad = EPAD - N_EDGES
    src_p = jnp.concatenate([src, jnp.zeros((pad,), jnp.int32)])
    dst_p = jnp.concatenate(
        [dst, N_NODES + (jnp.arange(pad, dtype=jnp.int32) % 128)])
    e_p = jnp.concatenate([e, jnp.ones((pad,), jnp.int32)])
    stk = jnp.stack([src_p, dst_p, e_p])

    partials = _sc_gather_scatter(tbl, stk)
    return _sum_partials(partials)


# speedup vs baseline: 4.7298x; 4.7298x over previous
import functools

import jax
import jax.numpy as jnp
from jax import lax
from jax.experimental import pallas as pl
from jax.experimental.pallas import tpu as pltpu
from jax.experimental.pallas import tpu_sc as plsc

N_NODES = 10000
N_EDGES = 320000
D = 128

_info = plsc.get_sparse_core_info()
NC = _info.num_cores
NS = _info.num_subcores
L = _info.num_lanes
NW = NC * NS

CHUNK = 128
EPW = -(-N_EDGES // (NW * CHUNK)) * CHUNK
EPAD = EPW * NW
NCHUNK = EPW // CHUNK

ACC_ROWS = 10240
ZROWS = ACC_ROWS // NS
WB = N_NODES // NS


def _elu_body(x_ref, w_ref, o_ref):
    j = pl.program_id(1)
    y = x_ref[...] * w_ref[...]
    y = jnp.where(y > 0, y, jnp.exp(y) - 1.0)
    o_ref[...] = y * (1.0 + j.astype(jnp.float32))


def _build_table(graph_embedding, W):
    blk = 1000
    grid = (N_NODES // blk, 2)
    return pl.pallas_call(
        _elu_body,
        grid=grid,
        in_specs=[
            pl.BlockSpec((blk, D), lambda i, j: (i, 0)),
            pl.BlockSpec((1, D), lambda i, j: (0, 0)),
        ],
        out_specs=pl.BlockSpec((blk, D), lambda i, j: (j * (N_NODES // blk) + i, 0)),
        out_shape=jax.ShapeDtypeStruct((2 * N_NODES, D), jnp.float32),
    )(graph_embedding, W)


_mesh = plsc.VectorSubcoreMesh(core_axis_name="c", subcore_axis_name="s")


@functools.partial(
    pl.kernel,
    mesh=_mesh,
    out_type=jax.ShapeDtypeStruct((NC, ACC_ROWS, D), jnp.float32),
    scratch_types=[
        pltpu.VMEM((3, CHUNK), jnp.int32),
        pltpu.VMEM((CHUNK,), jnp.int32),
        pltpu.VMEM((CHUNK, D), jnp.float32),
        pltpu.VMEM((16, D), jnp.float32),
        pltpu.VMEM_SHARED((ACC_ROWS, D), jnp.float32),
        pltpu.SemaphoreType.DMA,
    ],
)
def _sc_gather_scatter(tbl_hbm, stk_hbm, out_hbm,
                       stk_v, gidx_v, rows_v, zb_v, acc_sh, sem):
    c = lax.axis_index("c")
    s = lax.axis_index("s")

    zero16 = jnp.zeros((L,), jnp.float32)
    for r in range(16):
        for k in range(D // L):
            zb_v[r, pl.ds(k * L, L)] = zero16
    base_row = s * ZROWS
    for t in range(ZROWS // 16):
        pltpu.sync_copy(zb_v, acc_sh.at[pl.ds(base_row + t * 16, 16)])
    plsc.subcore_barrier()

    w = c * NS + s
    ebase = w * EPW

    def body(j, carry):
        pltpu.sync_copy(stk_hbm.at[:, pl.ds(ebase + j * CHUNK, CHUNK)], stk_v)
        for k in range(CHUNK // L):
            sl = pl.ds(k * L, L)
            s16 = stk_v[0, sl]
            e16 = stk_v[2, sl]
            gidx_v[sl] = s16 + (1 - (e16 & 1)) * N_NODES
        pltpu.async_copy(tbl_hbm.at[gidx_v], rows_v, sem).wait()
        pltpu.sync_copy(rows_v, acc_sh.at[stk_v.at[1]], add=True)
        return carry

    lax.fori_loop(0, NCHUNK, body, 0)
    plsc.subcore_barrier()
    pltpu.sync_copy(acc_sh.at[pl.ds(s * ZROWS, ZROWS)],
                    out_hbm.at[c, pl.ds(s * ZROWS, ZROWS)])


def _add_body(p_ref, o_ref):
    o_ref[...] = p_ref[0] + p_ref[1]


def _sum_partials(partials):
    blk = 1000
    return pl.pallas_call(
        _add_body,
        grid=(N_NODES // blk,),
        in_specs=[pl.BlockSpec((2, blk, D), lambda i: (0, i, 0))],
        out_specs=pl.BlockSpec((blk, D), lambda i: (i, 0)),
        out_shape=jax.ShapeDtypeStruct((N_NODES, D), jnp.float32),
    )(partials)


def kernel(graph_embedding, edge_index, e_feat, W):
    tbl = _build_table(graph_embedding, W)

    src = edge_index[0].astype(jnp.int32)
    dst = edge_index[1].astype(jnp.int32)
    e = e_feat.astype(jnp.int32)
    pad = EPAD - N_EDGES
    src_p = jnp.concatenate([src, jnp.zeros((pad,), jnp.int32)])
    dst_p = jnp.concatenate(
        [dst, N_NODES + (jnp.arange(pad, dtype=jnp.int32) % 128)])
    e_p = jnp.concatenate([e, jnp.ones((pad,), jnp.int32)])
    stk = jnp.stack([src_p, dst_p, e_p])

    partials = _sc_gather_scatter(tbl, stk)
    return _sum_partials(partials)
